# Initial kernel scaffold; baseline (speedup 1.0000x reference)
import jax
import jax.numpy as jnp
from jax.experimental import pallas as pl


def kernel(input, target_gram):
    raise NotImplementedError("write your pallas kernel here")


# speedup vs baseline: 18.4195x; 18.4195x over previous
import functools

import jax
import jax.numpy as jnp
from jax.experimental import pallas as pl

_TOPK_FRAC = 0.05


def _threshold_kernel(x_ref, t_ref, *, kn):
    x = x_ref[...]
    u = jax.lax.bitcast_convert_type(x, jnp.int32) & jnp.int32(0x7FFFFFFF)

    def body(i, t):
        cand = t | (jnp.int32(1) << (jnp.int32(30) - i))
        cnt = jnp.sum((u >= cand).astype(jnp.int32), axis=1, keepdims=True)
        return jnp.where(cnt >= kn, cand, t)

    t0 = jnp.zeros((x.shape[0], 1), jnp.int32)
    t = jax.lax.fori_loop(0, 31, body, t0)
    t_ref[0, 0, :] = t[:, 0]


def _gram_kernel(x_ref, t_ref, o_ref):
    n = pl.program_id(1)
    x = x_ref[0]
    u = jax.lax.bitcast_convert_type(x, jnp.int32) & jnp.int32(0x7FFFFFFF)
    t = t_ref[0]
    xm = jnp.where(u >= t, x, 0.0)
    g = jax.lax.dot_general(xm, xm, (((1,), (1,)), ((), ())),
                            preferred_element_type=jnp.float32)

    @pl.when(n == 0)
    def _init():
        o_ref[0] = g

    @pl.when(n != 0)
    def _acc():
        o_ref[0] += g


def _loss_kernel(g_ref, tg_ref, o_ref, *, inv_scale, loss_scale):
    g = g_ref[...] * inv_scale
    d = tg_ref[...] - g
    o_ref[...] = jnp.reshape(jnp.sum(d * d) * loss_scale, (1, 1))


def kernel(input, target_gram):
    b, c, h, w = input.shape
    n = h * w
    kn = max(1, int(_TOPK_FRAC * n))
    rows = b * c

    rb = 8
    thr = pl.pallas_call(
        functools.partial(_threshold_kernel, kn=kn),
        grid=(rows // rb,),
        in_specs=[pl.BlockSpec((rb, n), lambda i: (i, 0))],
        out_specs=pl.BlockSpec((1, 1, rb), lambda i: (i, 0, 0)),
        out_shape=jax.ShapeDtypeStruct((rows // rb, 1, rb), jnp.int32),
    )(input.reshape(rows, n))

    chunk = min(4096, n)
    nchunks = n // chunk
    graw = pl.pallas_call(
        _gram_kernel,
        grid=(b, nchunks),
        in_specs=[
            pl.BlockSpec((1, c, chunk), lambda bi, ni: (bi, 0, ni)),
            pl.BlockSpec((1, c, 1), lambda bi, ni: (bi, 0, 0)),
        ],
        out_specs=pl.BlockSpec((1, c, c), lambda bi, ni: (bi, 0, 0)),
        out_shape=jax.ShapeDtypeStruct((b, c, c), jnp.float32),
    )(input.reshape(b, c, n), thr.reshape(b, c, 1))

    loss2d = pl.pallas_call(
        functools.partial(
            _loss_kernel,
            inv_scale=1.0 / (b * c * n),
            loss_scale=1000000000.0 / (b * c * c),
        ),
        in_specs=[
            pl.BlockSpec((b * c, c), lambda: (0, 0)),
            pl.BlockSpec((b * c, c), lambda: (0, 0)),
        ],
        out_specs=pl.BlockSpec((1, 1), lambda: (0, 0)),
        out_shape=jax.ShapeDtypeStruct((1, 1), jnp.float32),
    )(graw.reshape(b * c, c), target_gram.reshape(b * c, c))

    return (input, loss2d[0, 0])


# speedup vs baseline: 18.5928x; 1.0094x over previous
import functools

import jax
import jax.numpy as jnp
from jax.experimental import pallas as pl

_TOPK_FRAC = 0.05


def _threshold_kernel(x_ref, t_ref, *, kn):
    x = x_ref[...]
    u = jax.lax.bitcast_convert_type(x, jnp.int32) & jnp.int32(0x7FFFFFFF)

    t = jnp.zeros((x.shape[0], 1), jnp.int32)
    for bit in range(30, -1, -1):
        cand = t | jnp.int32(1 << bit)
        cnt = jnp.sum((u >= cand).astype(jnp.int32), axis=1, keepdims=True)
        t = jnp.where(cnt >= kn, cand, t)
    t_ref[0, 0, :] = t[:, 0]


def _gram_kernel(x_ref, t_ref, o_ref):
    n = pl.program_id(1)
    x = x_ref[0]
    u = jax.lax.bitcast_convert_type(x, jnp.int32) & jnp.int32(0x7FFFFFFF)
    t = t_ref[0]
    xm = jnp.where(u >= t, x, 0.0)
    g = jax.lax.dot_general(xm, xm, (((1,), (1,)), ((), ())),
                            preferred_element_type=jnp.float32)

    @pl.when(n == 0)
    def _init():
        o_ref[0] = g

    @pl.when(n != 0)
    def _acc():
        o_ref[0] += g


def _loss_kernel(g_ref, tg_ref, o_ref, *, inv_scale, loss_scale):
    g = g_ref[...] * inv_scale
    d = tg_ref[...] - g
    o_ref[...] = jnp.reshape(jnp.sum(d * d) * loss_scale, (1, 1))


def kernel(input, target_gram):
    b, c, h, w = input.shape
    n = h * w
    kn = max(1, int(_TOPK_FRAC * n))
    rows = b * c

    rb = 8
    thr = pl.pallas_call(
            functools.partial(_threshold_kernel, kn=kn),
            grid=(rows // rb,),
            in_specs=[pl.BlockSpec((rb, n), lambda i: (i, 0))],
            out_specs=pl.BlockSpec((1, 1, rb), lambda i: (i, 0, 0)),
            out_shape=jax.ShapeDtypeStruct((rows // rb, 1, rb), jnp.int32),
        )(input.reshape(rows, n))

    chunk = min(4096, n)
    nchunks = n // chunk
    graw = pl.pallas_call(
        _gram_kernel,
        grid=(b, nchunks),
        in_specs=[
            pl.BlockSpec((1, c, chunk), lambda bi, ni: (bi, 0, ni)),
            pl.BlockSpec((1, c, 1), lambda bi, ni: (bi, 0, 0)),
        ],
        out_specs=pl.BlockSpec((1, c, c), lambda bi, ni: (bi, 0, 0)),
        out_shape=jax.ShapeDtypeStruct((b, c, c), jnp.float32),
    )(input.reshape(b, c, n), thr.reshape(b, c, 1))

    loss2d = pl.pallas_call(
        functools.partial(
            _loss_kernel,
            inv_scale=1.0 / (b * c * n),
            loss_scale=1000000000.0 / (b * c * c),
        ),
        in_specs=[
            pl.BlockSpec((b * c, c), lambda: (0, 0)),
            pl.BlockSpec((b * c, c), lambda: (0, 0)),
        ],
        out_specs=pl.BlockSpec((1, 1), lambda: (0, 0)),
        out_shape=jax.ShapeDtypeStruct((1, 1), jnp.float32),
    )(graw.reshape(b * c, c), target_gram.reshape(b * c, c))

    return (input, loss2d[0, 0])
